# baseline (speedup 1.0000x reference)
import functools

import jax
import jax.numpy as jnp
from jax import lax
from jax.experimental import pallas as pl
from jax.experimental.pallas import tpu as pltpu
from jax.experimental.pallas import tpu_sc as plsc

_N = 10000
_NPAD = 10240
_E = 320000
_NC, _NS = 2, 16
_NW = _NC * _NS
_EPW = _E // _NW
_K = 80
_CH = _EPW // _K
_D = 16
_RS = _NPAD // _NS
_G = 5
_NB = 4

_mesh = plsc.VectorSubcoreMesh(
    core_axis_name="c", subcore_axis_name="s",
    num_cores=_NC, num_subcores=_NS)
_SC_PARAMS = pltpu.CompilerParams(use_tc_tiling_on_sc=False)


@functools.partial(
    pl.kernel,
    out_type=jax.ShapeDtypeStruct((_NC, _NPAD), jnp.float32),
    mesh=_mesh, compiler_params=_SC_PARAMS,
    scratch_types=[
        pltpu.VMEM((_CH, _K), jnp.int32),
        pltpu.VMEM((_K,), jnp.float32),
        pltpu.VMEM_SHARED((_NPAD,), jnp.float32),
        pltpu.SemaphoreType.DMA,
    ],
)
def _sc_degree(ei_hbm, zero_hbm, one_hbm, out_hbm, idx_v, ones_v, deg_sp, sem):
    c = lax.axis_index("c")
    s = lax.axis_index("s")
    wid = s * _NC + c
    pltpu.sync_copy(zero_hbm.at[pl.ds(s * _RS, _RS)],
                    deg_sp.at[pl.ds(s * _RS, _RS)])
    pltpu.sync_copy(one_hbm, ones_v)
    pltpu.sync_copy(ei_hbm.at[1, wid], idx_v)
    plsc.subcore_barrier()

    def body(g, carry):
        for b in range(_G):
            pltpu.async_copy(ones_v, deg_sp.at[idx_v.at[g * _G + b]], sem,
                             add=True)
        for b in range(_G):
            pltpu.make_async_copy(ones_v, deg_sp.at[idx_v.at[g * _G + b]],
                                  sem).wait()
        return carry

    lax.fori_loop(0, _CH // _G, body, 0)
    plsc.subcore_barrier()
    pltpu.sync_copy(deg_sp.at[pl.ds(s * _RS, _RS)],
                    out_hbm.at[c, pl.ds(s * _RS, _RS)])


@functools.partial(
    pl.kernel,
    out_type=jax.ShapeDtypeStruct((_NC, _NPAD, _D), jnp.float32),
    mesh=_mesh, compiler_params=_SC_PARAMS,
    scratch_types=(
        [pltpu.VMEM((_CH, _K), jnp.int32),
         pltpu.VMEM((_CH, _K), jnp.int32)]
        + [pltpu.VMEM((_K, _D), jnp.float32)] * _NB
        + [pltpu.VMEM_SHARED((_NPAD, _D), jnp.float32),
           pltpu.VMEM_SHARED((_NPAD, _D), jnp.float32)]
        + [pltpu.SemaphoreType.DMA] * (2 * _NB)
    ),
)
def _sc_segsum(g_hbm, ei_hbm, zero_hbm, out_hbm, idx_s_v, idx_d_v,
               r0, r1, r2, r3, g_sp, acc_sp,
               g0, g1_, g2_, g3, s0, s1, s2, s3):
    c = lax.axis_index("c")
    s = lax.axis_index("s")
    wid = s * _NC + c
    rows = (r0, r1, r2, r3)
    gsem = (g0, g1_, g2_, g3)
    ssem = (s0, s1, s2, s3)
    pltpu.sync_copy(zero_hbm.at[pl.ds(s * _RS, _RS)],
                    acc_sp.at[pl.ds(s * _RS, _RS)])
    pltpu.sync_copy(g_hbm.at[pl.ds(s * _RS, _RS)],
                    g_sp.at[pl.ds(s * _RS, _RS)])
    pltpu.sync_copy(ei_hbm.at[0, wid], idx_s_v)
    pltpu.sync_copy(ei_hbm.at[1, wid], idx_d_v)
    plsc.subcore_barrier()

    def gather(ch, b):
        pltpu.async_copy(g_sp.at[idx_s_v.at[ch]], rows[b], gsem[b])

    def gather_wait(ch, b):
        pltpu.make_async_copy(g_sp.at[idx_s_v.at[ch]], rows[b], gsem[b]).wait()

    def scatter(ch, b):
        pltpu.async_copy(rows[b], acc_sp.at[idx_d_v.at[ch]], ssem[b], add=True)

    def scatter_wait(ch, b):
        pltpu.make_async_copy(rows[b], acc_sp.at[idx_d_v.at[ch]],
                              ssem[b]).wait()

    for b in range(_NB):
        gather(b, b)

    def body(p, carry):
        c0 = 4 * p
        for b in range(_NB):
            gather_wait(c0 + b, b)
            scatter(c0 + b, b)
        for b in range(_NB):
            n = c0 + 4 + b

            @pl.when(n < _CH)
            def _():
                scatter_wait(c0 + b, b)
                gather(n, b)

        return carry

    lax.fori_loop(0, _CH // _NB, body, 0)
    gather_wait(_CH - 1, 0)
    scatter(_CH - 1, 0)
    scatter_wait(_CH - 1, 0)
    for b in range(1, _NB):
        scatter_wait(_CH - 5 + b, b)
    plsc.subcore_barrier()
    pltpu.sync_copy(acc_sp.at[pl.ds(s * _RS, _RS)],
                    out_hbm.at[c, pl.ds(s * _RS, _RS)])


def _tc1a_body(x_ref, w_ref, h_ref):
    h_ref[_N:, :] = jnp.zeros((_NPAD - _N, _D), jnp.float32)
    h_ref[:_N, :] = jnp.dot(x_ref[...], w_ref[...],
                            preferred_element_type=jnp.float32)


def _tc1b_body(h_ref, deg_ref, g_ref, dinv_ref):
    deg = deg_ref[0] + deg_ref[1]
    dinv = jnp.where(deg > 0, lax.rsqrt(jnp.maximum(deg, 1e-12)), 0.0)
    g_ref[...] = h_ref[...] * dinv
    dinv_ref[...] = dinv


def _tc2_body(acc_ref, dinv_ref, b_ref, g_ref):
    dinv = dinv_ref[...]
    u = jnp.maximum(dinv * (acc_ref[0] + acc_ref[1]) + b_ref[...], 0.0)
    g_ref[...] = dinv * u


def _tc3_body(acc_ref, dinv_ref, w_ref, b_ref, o_ref):
    t = dinv_ref[...] * (acc_ref[0] + acc_ref[1])
    o = jnp.dot(t, w_ref[...], preferred_element_type=jnp.float32) + b_ref[...]
    o = jnp.maximum(o, 0.0)
    m = jnp.max(o, axis=1, keepdims=True)
    sh = o - m
    lse = jnp.log(jnp.sum(jnp.exp(sh), axis=1, keepdims=True))
    o_ref[...] = sh - lse


_BLK = 1024
_BLK3 = 1000


def kernel(x, edge_index, W1, b1, W2, b2):
    f32 = jnp.float32
    ei3 = edge_index.astype(jnp.int32).reshape(2, _NW, _CH, _K)
    ei3 = lax.optimization_barrier(ei3)
    zeros1 = jnp.zeros((_NPAD,), f32)
    zeros2 = jnp.zeros((_NPAD, _D), f32)
    ones = jnp.ones((_K,), f32)

    deg2 = _sc_degree(ei3, zeros1, ones)

    h1 = pl.pallas_call(
        _tc1a_body,
        out_shape=jax.ShapeDtypeStruct((_NPAD, _D), f32),
    )(x, W1)

    g1, dinv = pl.pallas_call(
        _tc1b_body,
        grid=(_NPAD // _BLK,),
        in_specs=[pl.BlockSpec((_BLK, _D), lambda i: (i, 0)),
                  pl.BlockSpec((_NC, _BLK, 1), lambda i: (0, i, 0))],
        out_specs=[pl.BlockSpec((_BLK, _D), lambda i: (i, 0)),
                   pl.BlockSpec((_BLK, 1), lambda i: (i, 0))],
        out_shape=[jax.ShapeDtypeStruct((_NPAD, _D), f32),
                   jax.ShapeDtypeStruct((_NPAD, 1), f32)],
    )(h1, deg2.reshape(_NC, _NPAD, 1))

    acc1 = _sc_segsum(g1, ei3, zeros2)

    g2 = pl.pallas_call(
        _tc2_body,
        grid=(_NPAD // _BLK,),
        in_specs=[pl.BlockSpec((_NC, _BLK, _D), lambda i: (0, i, 0)),
                  pl.BlockSpec((_BLK, 1), lambda i: (i, 0)),
                  pl.BlockSpec((1, _D), lambda i: (0, 0))],
        out_specs=pl.BlockSpec((_BLK, _D), lambda i: (i, 0)),
        out_shape=jax.ShapeDtypeStruct((_NPAD, _D), f32),
    )(acc1, dinv, b1.reshape(1, _D))

    acc2 = _sc_segsum(g2, ei3, zeros2)

    dout = W2.shape[1]
    out = pl.pallas_call(
        _tc3_body,
        grid=(_N // _BLK3,),
        in_specs=[pl.BlockSpec((_NC, _BLK3, _D), lambda i: (0, i, 0)),
                  pl.BlockSpec((_BLK3, 1), lambda i: (i, 0)),
                  pl.BlockSpec((_D, dout), lambda i: (0, 0)),
                  pl.BlockSpec((1, dout), lambda i: (0, 0))],
        out_specs=pl.BlockSpec((_BLK3, dout), lambda i: (i, 0)),
        out_shape=jax.ShapeDtypeStruct((_N, dout), f32),
    )(acc2, dinv, W2, b2.reshape(1, dout))

    return out
